# Initial kernel scaffold; baseline (speedup 1.0000x reference)
import jax
import jax.numpy as jnp
from jax.experimental import pallas as pl


def kernel(seq, hidden, table):
    raise NotImplementedError("write your pallas kernel here")


# speedup vs baseline: 4.0810x; 4.0810x over previous
import jax
import jax.numpy as jnp
from jax import lax
from jax.experimental import pallas as pl
from jax.experimental.pallas import tpu as pltpu
from jax.experimental.pallas import tpu_sc as plsc

NUM_CORES = 2
NUM_SUBCORES = 16
NUM_WORKERS = NUM_CORES * NUM_SUBCORES

GROUP = 128


def _gather_body(idx_hbm, table_hbm, out_hbm, idx_v, rows_v, sem):
    c = lax.axis_index("c")
    s = lax.axis_index("s")
    wid = s * NUM_CORES + c
    n_per_w = idx_hbm.shape[0] // NUM_WORKERS
    groups_per_w = n_per_w // GROUP
    base = wid * n_per_w
    pltpu.sync_copy(idx_hbm.at[pl.ds(base, n_per_w)], idx_v)

    def step(j, carry):
        idx_g = idx_v.at[pl.ds(j * GROUP, GROUP)]
        pltpu.async_copy(table_hbm.at[idx_g], rows_v, sem).wait()
        pltpu.sync_copy(rows_v, out_hbm.at[pl.ds(base + j * GROUP, GROUP)])
        return carry

    lax.fori_loop(0, groups_per_w, step, 0)


def kernel(seq, hidden, table):
    B, L = seq.shape
    V, D = table.shape
    N = B * L
    idx = seq.reshape(N)
    n_per_w = N // NUM_WORKERS

    mesh = plsc.VectorSubcoreMesh(core_axis_name="c", subcore_axis_name="s")
    out = pl.kernel(
        _gather_body,
        out_type=jax.ShapeDtypeStruct((N, D), jnp.float32),
        mesh=mesh,
        scratch_types=[
            pltpu.VMEM((n_per_w,), jnp.int32),
            pltpu.VMEM((GROUP, D), jnp.float32),
            pltpu.SemaphoreType.DMA,
        ],
        compiler_params=pltpu.CompilerParams(use_tc_tiling_on_sc=False),
    )(idx, table)
    return out.reshape(B, L, D)


# speedup vs baseline: 4.5932x; 1.1255x over previous
import jax
import jax.numpy as jnp
from jax import lax
from jax.experimental import pallas as pl
from jax.experimental.pallas import tpu as pltpu
from jax.experimental.pallas import tpu_sc as plsc

NUM_CORES = 2
NUM_SUBCORES = 16
NUM_WORKERS = NUM_CORES * NUM_SUBCORES

GROUP = 128


GPB = 5
NBUF = 2


def _gather_body(idx_hbm, table_hbm, out_hbm, idx_v,
                 rows0, rows1, gsem0, gsem1, wsem0, wsem1):
    c = lax.axis_index("c")
    s = lax.axis_index("s")
    wid = s * NUM_CORES + c
    n_per_w = idx_hbm.shape[0] // NUM_WORKERS
    groups_per_w = n_per_w // GROUP
    n_blocks = groups_per_w // GPB
    block_rows = GPB * GROUP
    base = wid * n_per_w
    pltpu.sync_copy(idx_hbm.at[pl.ds(base, n_per_w)], idx_v)

    bufs = (rows0, rows1)
    gsems = (gsem0, gsem1)
    wsems = (wsem0, wsem1)
    wb = [None, None]

    for b in range(n_blocks):
        k = b % NBUF
        if wb[k] is not None:
            wb[k].wait()
        gd = []
        for g in range(GPB):
            jg = b * GPB + g
            idx_g = idx_v.at[pl.ds(jg * GROUP, GROUP)]
            dst = bufs[k].at[pl.ds(g * GROUP, GROUP)]
            gd.append(pltpu.async_copy(table_hbm.at[idx_g], dst, gsems[k]))
        for d in gd:
            d.wait()
        wb[k] = pltpu.async_copy(
            bufs[k], out_hbm.at[pl.ds(base + b * block_rows, block_rows)],
            wsems[k])
    for k in range(NBUF):
        if wb[k] is not None:
            wb[k].wait()


def kernel(seq, hidden, table):
    B, L = seq.shape
    V, D = table.shape
    N = B * L
    idx = seq.reshape(N)
    n_per_w = N // NUM_WORKERS

    mesh = plsc.VectorSubcoreMesh(core_axis_name="c", subcore_axis_name="s")
    out = pl.kernel(
        _gather_body,
        out_type=jax.ShapeDtypeStruct((N, D), jnp.float32),
        mesh=mesh,
        scratch_types=[
            pltpu.VMEM((n_per_w,), jnp.int32),
            pltpu.VMEM((GPB * GROUP, D), jnp.float32),
            pltpu.VMEM((GPB * GROUP, D), jnp.float32),
            pltpu.SemaphoreType.DMA,
            pltpu.SemaphoreType.DMA,
            pltpu.SemaphoreType.DMA,
            pltpu.SemaphoreType.DMA,
        ],
        compiler_params=pltpu.CompilerParams(use_tc_tiling_on_sc=False),
    )(idx, table)
    return out.reshape(B, L, D)
